# Initial kernel scaffold; baseline (speedup 1.0000x reference)
import jax
import jax.numpy as jnp
from jax.experimental import pallas as pl


def kernel(x, W1, b1, W2, b2, keep_ratio):
    raise NotImplementedError("write your pallas kernel here")


# speedup vs baseline: 1.4006x; 1.4006x over previous
import functools

import jax
import jax.numpy as jnp
from jax import lax
from jax.experimental import pallas as pl


def _stats_kernel(x_ref, o_ref):
    o_ref[...] = jnp.mean(x_ref[...], axis=1, keepdims=True)


def _mlp_kernel(stats_ref, w1_ref, b1_ref, w2_ref, b2_ref, scores_ref):
    cs = stats_ref[...][:, 0, :]
    h = jnp.dot(cs, w1_ref[...], preferred_element_type=jnp.float32)
    h = jnp.maximum(h + b1_ref[...], 0.0)
    s = jnp.dot(h, w2_ref[...], preferred_element_type=jnp.float32)
    scores_ref[...] = (s + b2_ref[...])[:, None, :]


def _apply_kernel(scores_ref, x_ref, out_ref, maske_ref, *, k):
    row = scores_ref[...][:, 0, :]
    C = row.shape[1]
    ones_row = jnp.ones((1, C), jnp.float32)
    colmat = lax.dot_general(row, ones_row, (((0,), (0,)), ((), ())),
                             preferred_element_type=jnp.float32)
    rowmat = jnp.broadcast_to(row, (C, C))
    i_idx = lax.broadcasted_iota(jnp.int32, (C, C), 0)
    c_idx = lax.broadcasted_iota(jnp.int32, (C, C), 1)
    beats = ((colmat > rowmat) | ((colmat == rowmat) & (i_idx < c_idx)))
    rank = lax.dot_general(ones_row, beats.astype(jnp.float32),
                           (((1,), (0,)), ((), ())),
                           preferred_element_type=jnp.float32)
    mrow = (rank < float(k)).astype(jnp.float32)
    me = jnp.broadcast_to(mrow[:, None, :], out_ref.shape)
    out_ref[...] = x_ref[...] * me
    maske_ref[...] = me


def kernel(x, W1, b1, W2, b2, keep_ratio):
    B, N, C = x.shape
    k = max(1, C // 2)

    stats = pl.pallas_call(
        _stats_kernel,
        grid=(B,),
        in_specs=[pl.BlockSpec((1, N, C), lambda b: (b, 0, 0))],
        out_specs=pl.BlockSpec((1, 1, C), lambda b: (b, 0, 0)),
        out_shape=jax.ShapeDtypeStruct((B, 1, C), jnp.float32),
    )(x)

    scores = pl.pallas_call(
        _mlp_kernel,
        out_shape=jax.ShapeDtypeStruct((B, 1, C), jnp.float32),
    )(stats, W1, b1.reshape(1, -1), W2, b2.reshape(1, -1))

    out, mask_e = pl.pallas_call(
        functools.partial(_apply_kernel, k=k),
        grid=(B,),
        in_specs=[
            pl.BlockSpec((1, 1, C), lambda b: (b, 0, 0)),
            pl.BlockSpec((1, N, C), lambda b: (b, 0, 0)),
        ],
        out_specs=[
            pl.BlockSpec((1, N, C), lambda b: (b, 0, 0)),
            pl.BlockSpec((1, N, C), lambda b: (b, 0, 0)),
        ],
        out_shape=[
            jax.ShapeDtypeStruct((B, N, C), jnp.float32),
            jax.ShapeDtypeStruct((B, N, C), jnp.float32),
        ],
    )(scores, x)
    return (out, mask_e)
